# Initial kernel scaffold; baseline (speedup 1.0000x reference)
import jax
import jax.numpy as jnp
from jax.experimental import pallas as pl


def kernel(coords, feats, q_w1, q_b1, k_w1, k_b1, v_w1, v_b1, conv_w1, conv_b1, bn_g1, bn_b1, q_w2, q_b2, k_w2, k_b2, v_w2, v_b2, conv_w2, conv_b2, bn_g2, bn_b2):
    raise NotImplementedError("write your pallas kernel here")


# speedup vs baseline: 4.8175x; 4.8175x over previous
import functools

import jax
import jax.numpy as jnp
from jax import lax
from jax.experimental import pallas as pl
from jax.experimental.pallas import tpu as pltpu
from jax.experimental.pallas import tpu_sc as plsc

N = 10000
D = 128
K = 16
NPAD = 10240
KNN_R = 80
ATT_R = 400
ROW_R = 2000
NWORK = 32
GCHUNK = 200


def _knn_body(cpad_ref, ct_ref, idx_ref, sq_ref):
    i = pl.program_id(0)

    @pl.when(i == 0)
    def _():
        ct = ct_ref[...]
        sq_ref[...] = jnp.sum(ct * ct, axis=0, keepdims=True)

    crow = cpad_ref[...]
    dot = lax.dot_general(crow, ct_ref[...],
                          dimension_numbers=(((1,), (0,)), ((), ())),
                          preferred_element_type=jnp.float32)
    sq_r = jnp.sum(crow * crow, axis=1, keepdims=True)
    d2 = sq_r + sq_ref[...] - 2.0 * dot
    col = lax.broadcasted_iota(jnp.int32, d2.shape, 1)
    inf = jnp.float32(jnp.inf)
    d2 = jnp.where(col >= N, inf, d2)

    picks = []
    big = jnp.int32(2**30)
    for _ in range(K):
        m = jnp.min(d2, axis=1, keepdims=True)
        cand = jnp.where(d2 == m, col, big)
        amin = jnp.min(cand, axis=1, keepdims=True)
        picks.append(amin)
        d2 = jnp.where(col == amin, inf, d2)
    idx_ref[...] = jnp.concatenate(picks, axis=1)


def _knn(coords):
    cpad = jnp.zeros((NPAD, 128), jnp.float32).at[:N, :3].set(coords)
    ct = cpad.T
    grid = N // KNN_R
    return pl.pallas_call(
        _knn_body,
        grid=(grid,),
        in_specs=[
            pl.BlockSpec((KNN_R, 128), lambda i: (i, 0)),
            pl.BlockSpec((128, NPAD), lambda i: (0, 0)),
        ],
        out_specs=pl.BlockSpec((KNN_R, K), lambda i: (i, 0)),
        out_shape=jax.ShapeDtypeStruct((N, K), jnp.int32),
        scratch_shapes=[pltpu.VMEM((1, NPAD), jnp.float32)],
    )(cpad, ct)


def _qkv_body(x_ref, qw_ref, qb_ref, kw_ref, kb_ref, vw_ref, vb_ref,
              q_ref, kv_ref):
    x = x_ref[...]
    dn = (((1,), (1,)), ((), ()))
    q = lax.dot_general(x, qw_ref[...], dn,
                        preferred_element_type=jnp.float32) + qb_ref[...]
    k = lax.dot_general(x, kw_ref[...], dn,
                        preferred_element_type=jnp.float32) + kb_ref[...]
    v = lax.dot_general(x, vw_ref[...], dn,
                        preferred_element_type=jnp.float32) + vb_ref[...]
    q_ref[...] = q
    kv_ref[...] = jnp.concatenate([k, v], axis=1)


def _qkv(x, qw, qb, kw, kb, vw, vb):
    grid = N // ROW_R
    row = pl.BlockSpec((ROW_R, 128), lambda i: (i, 0))
    wsp = pl.BlockSpec((128, 128), lambda i: (0, 0))
    bsp = pl.BlockSpec((1, 128), lambda i: (0, 0))
    return pl.pallas_call(
        _qkv_body,
        grid=(grid,),
        in_specs=[row, wsp, bsp, wsp, bsp, wsp, bsp],
        out_specs=[row, pl.BlockSpec((ROW_R, 256), lambda i: (i, 0))],
        out_shape=[jax.ShapeDtypeStruct((N, 128), jnp.float32),
                   jax.ShapeDtypeStruct((N, 256), jnp.float32)],
    )(x, qw, qb.reshape(1, 128), kw, kb.reshape(1, 128),
      vw, vb.reshape(1, 128))


def _gather_body(idx_hbm, table_hbm, out_hbm, idx_v, rows_v, sem):
    wid = lax.axis_index("s") * 2 + lax.axis_index("c")
    per_w = (N * K) // NWORK
    base = wid * per_w
    pltpu.sync_copy(idx_hbm.at[pl.ds(base, per_w)], idx_v)

    def chunk(c, carry):
        off = c * GCHUNK
        pltpu.async_copy(table_hbm.at[idx_v.at[pl.ds(off, GCHUNK)]],
                         rows_v, sem).wait()
        pltpu.sync_copy(rows_v, out_hbm.at[pl.ds(base + off, GCHUNK)])
        return carry

    lax.fori_loop(0, per_w // GCHUNK, chunk, 0)


def _sc_gather(idx_flat, table):
    per_w = (N * K) // NWORK
    mesh = plsc.VectorSubcoreMesh(core_axis_name="c", subcore_axis_name="s")
    fn = pl.kernel(
        _gather_body,
        mesh=mesh,
        out_type=jax.ShapeDtypeStruct((N * K, 256), jnp.float32),
        scratch_types=[
            pltpu.VMEM((per_w,), jnp.int32),
            pltpu.VMEM((GCHUNK, 256), jnp.float32),
            pltpu.SemaphoreType.DMA,
        ],
    )
    return fn(idx_flat, table)


def _attn_body(nbr_ref, q_ref, x_ref, cw_ref, cb_ref, h_ref, st_ref):
    i = pl.program_id(0)
    nbr = nbr_ref[...]
    nk = nbr[:, :, :128]
    nv = nbr[:, :, 128:]
    q = q_ref[...]
    logits = jnp.sum(nk * q[:, None, :], axis=2)
    logits = logits * jnp.float32(1.0 / jnp.sqrt(128.0))
    m = jnp.max(logits, axis=1, keepdims=True)
    e = jnp.exp(logits - m)
    attn = e / jnp.sum(e, axis=1, keepdims=True)
    att = jnp.sum(attn[:, :, None] * nv, axis=1)
    h = lax.dot_general(x_ref[...] - att, cw_ref[...],
                        (((1,), (1,)), ((), ())),
                        preferred_element_type=jnp.float32) + cb_ref[...]
    h_ref[...] = h
    s1 = jnp.sum(h, axis=0, keepdims=True)
    s2 = jnp.sum(h * h, axis=0, keepdims=True)
    pad = jnp.zeros((6, 128), jnp.float32)
    upd = jnp.concatenate([s1, s2, pad], axis=0)

    @pl.when(i == 0)
    def _():
        st_ref[...] = jnp.zeros_like(st_ref)

    st_ref[...] += upd


def _attn(nbr, q, x, cw, cb):
    grid = N // ATT_R
    row = pl.BlockSpec((ATT_R, 128), lambda i: (i, 0))
    return pl.pallas_call(
        _attn_body,
        grid=(grid,),
        in_specs=[
            pl.BlockSpec((ATT_R, K, 256), lambda i: (i, 0, 0)),
            row, row,
            pl.BlockSpec((128, 128), lambda i: (0, 0)),
            pl.BlockSpec((1, 128), lambda i: (0, 0)),
        ],
        out_specs=[row, pl.BlockSpec((8, 128), lambda i: (0, 0))],
        out_shape=[jax.ShapeDtypeStruct((N, 128), jnp.float32),
                   jax.ShapeDtypeStruct((8, 128), jnp.float32)],
    )(nbr, q, x, cw, cb.reshape(1, 128))


def _bn_core(h, x, st_ref, g_ref, b_ref):
    inv_n = jnp.float32(1.0 / N)
    mu = st_ref[0:1, :] * inv_n
    var = st_ref[1:2, :] * inv_n - mu * mu
    hn = (h - mu) * lax.rsqrt(var + 1e-5) * g_ref[...] + b_ref[...]
    return x + jnp.maximum(hn, 0.0)


def _bn_qkv_body(h_ref, x_ref, st_ref, g_ref, b_ref,
                 qw_ref, qb_ref, kw_ref, kb_ref, vw_ref, vb_ref,
                 out_ref, q_ref, kv_ref):
    out = _bn_core(h_ref[...], x_ref[...], st_ref, g_ref, b_ref)
    out_ref[...] = out
    dn = (((1,), (1,)), ((), ()))
    q = lax.dot_general(out, qw_ref[...], dn,
                        preferred_element_type=jnp.float32) + qb_ref[...]
    k = lax.dot_general(out, kw_ref[...], dn,
                        preferred_element_type=jnp.float32) + kb_ref[...]
    v = lax.dot_general(out, vw_ref[...], dn,
                        preferred_element_type=jnp.float32) + vb_ref[...]
    q_ref[...] = q
    kv_ref[...] = jnp.concatenate([k, v], axis=1)


def _bn_qkv(h, x, st, g, b, qw, qb, kw, kb, vw, vb):
    grid = N // ROW_R
    row = pl.BlockSpec((ROW_R, 128), lambda i: (i, 0))
    wsp = pl.BlockSpec((128, 128), lambda i: (0, 0))
    bsp = pl.BlockSpec((1, 128), lambda i: (0, 0))
    ssp = pl.BlockSpec((8, 128), lambda i: (0, 0))
    return pl.pallas_call(
        _bn_qkv_body,
        grid=(grid,),
        in_specs=[row, row, ssp, bsp, bsp, wsp, bsp, wsp, bsp, wsp, bsp],
        out_specs=[row, row, pl.BlockSpec((ROW_R, 256), lambda i: (i, 0))],
        out_shape=[jax.ShapeDtypeStruct((N, 128), jnp.float32),
                   jax.ShapeDtypeStruct((N, 128), jnp.float32),
                   jax.ShapeDtypeStruct((N, 256), jnp.float32)],
    )(h, x, st, g.reshape(1, 128), b.reshape(1, 128),
      qw, qb.reshape(1, 128), kw, kb.reshape(1, 128), vw, vb.reshape(1, 128))


def _bn_final_body(h_ref, x_ref, st_ref, g_ref, b_ref, out_ref):
    out_ref[...] = _bn_core(h_ref[...], x_ref[...], st_ref, g_ref, b_ref)


def _bn_final(h, x, st, g, b):
    grid = N // ROW_R
    row = pl.BlockSpec((ROW_R, 128), lambda i: (i, 0))
    bsp = pl.BlockSpec((1, 128), lambda i: (0, 0))
    ssp = pl.BlockSpec((8, 128), lambda i: (0, 0))
    return pl.pallas_call(
        _bn_final_body,
        grid=(grid,),
        in_specs=[row, row, ssp, bsp, bsp],
        out_specs=row,
        out_shape=jax.ShapeDtypeStruct((N, 128), jnp.float32),
    )(h, x, st, g.reshape(1, 128), b.reshape(1, 128))


def kernel(coords, feats,
           q_w1, q_b1, k_w1, k_b1, v_w1, v_b1, conv_w1, conv_b1, bn_g1, bn_b1,
           q_w2, q_b2, k_w2, k_b2, v_w2, v_b2, conv_w2, conv_b2, bn_g2, bn_b2):
    idx = _knn(coords)
    idx_flat = idx.reshape(N * K)

    q1, kv1 = _qkv(feats, q_w1, q_b1, k_w1, k_b1, v_w1, v_b1)
    nbr1 = _sc_gather(idx_flat, kv1).reshape(N, K, 256)
    h1, st1 = _attn(nbr1, q1, feats, conv_w1, conv_b1)
    out1, q2, kv2 = _bn_qkv(h1, feats, st1, bn_g1, bn_b1,
                            q_w2, q_b2, k_w2, k_b2, v_w2, v_b2)

    nbr2 = _sc_gather(idx_flat, kv2).reshape(N, K, 256)
    h2, st2 = _attn(nbr2, q2, out1, conv_w2, conv_b2)
    return _bn_final(h2, out1, st2, bn_g2, bn_b2)


# speedup vs baseline: 7.8550x; 1.6305x over previous
import functools

import jax
import jax.numpy as jnp
from jax import lax
from jax.experimental import pallas as pl
from jax.experimental.pallas import tpu as pltpu
from jax.experimental.pallas import tpu_sc as plsc

N = 10000
D = 128
K = 16
NPAD = 10240
KNN_R = 80
ATT_R = 400
ROW_R = 2000
NWORK = 32
GCHUNK = 200


def _knn_body(cpad_ref, ct_ref, idx_ref, sq_ref):
    i = pl.program_id(0)

    @pl.when(i == 0)
    def _():
        ct = ct_ref[...]
        sq_ref[...] = jnp.sum(ct * ct, axis=0, keepdims=True)

    crow = cpad_ref[...]
    dot = lax.dot_general(crow, ct_ref[...],
                          dimension_numbers=(((1,), (0,)), ((), ())),
                          preferred_element_type=jnp.float32)
    sq_r = jnp.sum(crow * crow, axis=1, keepdims=True)
    d2 = sq_r + sq_ref[...] - 2.0 * dot
    col = lax.broadcasted_iota(jnp.int32, d2.shape, 1)
    inf = jnp.float32(jnp.inf)
    d2 = jnp.where(col >= N, inf, d2)

    C = 256
    T = 5
    G = NPAD // C
    cls = lax.broadcasted_iota(jnp.int32, (KNN_R, C), 1)
    mv = [jnp.full((KNN_R, C), inf, jnp.float32) for _ in range(T)]
    mi = [jnp.zeros((KNN_R, C), jnp.int32) for _ in range(T)]
    for g in range(G):
        x = lax.slice(d2, (0, g * C), (KNN_R, (g + 1) * C))
        xi = cls + jnp.int32(g * C)
        for t in range(T):
            c = x < mv[t]
            nm = jnp.where(c, x, mv[t])
            ni = jnp.where(c, xi, mi[t])
            x = jnp.where(c, mv[t], x)
            xi = jnp.where(c, mi[t], xi)
            mv[t] = nm
            mi[t] = ni
    cand = jnp.concatenate(mv, axis=1)
    cidx = jnp.concatenate(mi, axis=1)

    picks = []
    big = jnp.int32(2**30)
    for _ in range(K):
        m = jnp.min(cand, axis=1, keepdims=True)
        sel = jnp.where(cand == m, cidx, big)
        amin = jnp.min(sel, axis=1, keepdims=True)
        picks.append(amin)
        cand = jnp.where(sel == amin, inf, cand)
    idx_ref[...] = jnp.concatenate(picks, axis=1)


def _knn(coords):
    cpad = jnp.zeros((NPAD, 128), jnp.float32).at[:N, :3].set(coords)
    ct = cpad.T
    grid = N // KNN_R
    return pl.pallas_call(
        _knn_body,
        grid=(grid,),
        in_specs=[
            pl.BlockSpec((KNN_R, 128), lambda i: (i, 0)),
            pl.BlockSpec((128, NPAD), lambda i: (0, 0)),
        ],
        out_specs=pl.BlockSpec((KNN_R, K), lambda i: (i, 0)),
        out_shape=jax.ShapeDtypeStruct((N, K), jnp.int32),
        scratch_shapes=[pltpu.VMEM((1, NPAD), jnp.float32)],
    )(cpad, ct)


def _qkv_body(x_ref, qw_ref, qb_ref, kw_ref, kb_ref, vw_ref, vb_ref,
              q_ref, kv_ref):
    x = x_ref[...]
    dn = (((1,), (1,)), ((), ()))
    q = lax.dot_general(x, qw_ref[...], dn,
                        preferred_element_type=jnp.float32) + qb_ref[...]
    k = lax.dot_general(x, kw_ref[...], dn,
                        preferred_element_type=jnp.float32) + kb_ref[...]
    v = lax.dot_general(x, vw_ref[...], dn,
                        preferred_element_type=jnp.float32) + vb_ref[...]
    q_ref[...] = q
    kv_ref[...] = jnp.concatenate([k, v], axis=1)


def _qkv(x, qw, qb, kw, kb, vw, vb):
    grid = N // ROW_R
    row = pl.BlockSpec((ROW_R, 128), lambda i: (i, 0))
    wsp = pl.BlockSpec((128, 128), lambda i: (0, 0))
    bsp = pl.BlockSpec((1, 128), lambda i: (0, 0))
    return pl.pallas_call(
        _qkv_body,
        grid=(grid,),
        in_specs=[row, wsp, bsp, wsp, bsp, wsp, bsp],
        out_specs=[row, pl.BlockSpec((ROW_R, 256), lambda i: (i, 0))],
        out_shape=[jax.ShapeDtypeStruct((N, 128), jnp.float32),
                   jax.ShapeDtypeStruct((N, 256), jnp.float32)],
    )(x, qw, qb.reshape(1, 128), kw, kb.reshape(1, 128),
      vw, vb.reshape(1, 128))


def _gather_body(idx_hbm, table_hbm, out_hbm, idx_v, rows_v, sem):
    wid = lax.axis_index("s") * 2 + lax.axis_index("c")
    per_w = (N * K) // NWORK
    base = wid * per_w
    pltpu.sync_copy(idx_hbm.at[pl.ds(base, per_w)], idx_v)

    def chunk(c, carry):
        off = c * GCHUNK
        pltpu.async_copy(table_hbm.at[idx_v.at[pl.ds(off, GCHUNK)]],
                         rows_v, sem).wait()
        pltpu.sync_copy(rows_v, out_hbm.at[pl.ds(base + off, GCHUNK)])
        return carry

    lax.fori_loop(0, per_w // GCHUNK, chunk, 0)


def _sc_gather(idx_flat, table):
    per_w = (N * K) // NWORK
    mesh = plsc.VectorSubcoreMesh(core_axis_name="c", subcore_axis_name="s")
    fn = pl.kernel(
        _gather_body,
        mesh=mesh,
        out_type=jax.ShapeDtypeStruct((N * K, 256), jnp.float32),
        scratch_types=[
            pltpu.VMEM((per_w,), jnp.int32),
            pltpu.VMEM((GCHUNK, 256), jnp.float32),
            pltpu.SemaphoreType.DMA,
        ],
    )
    return fn(idx_flat, table)


def _attn_body(nbr_ref, q_ref, x_ref, cw_ref, cb_ref, h_ref, st_ref):
    i = pl.program_id(0)
    nbr = nbr_ref[...]
    nk = nbr[:, :, :128]
    nv = nbr[:, :, 128:]
    q = q_ref[...]
    logits = jnp.sum(nk * q[:, None, :], axis=2)
    logits = logits * jnp.float32(1.0 / jnp.sqrt(128.0))
    m = jnp.max(logits, axis=1, keepdims=True)
    e = jnp.exp(logits - m)
    attn = e / jnp.sum(e, axis=1, keepdims=True)
    att = jnp.sum(attn[:, :, None] * nv, axis=1)
    h = lax.dot_general(x_ref[...] - att, cw_ref[...],
                        (((1,), (1,)), ((), ())),
                        preferred_element_type=jnp.float32) + cb_ref[...]
    h_ref[...] = h
    s1 = jnp.sum(h, axis=0, keepdims=True)
    s2 = jnp.sum(h * h, axis=0, keepdims=True)
    pad = jnp.zeros((6, 128), jnp.float32)
    upd = jnp.concatenate([s1, s2, pad], axis=0)

    @pl.when(i == 0)
    def _():
        st_ref[...] = jnp.zeros_like(st_ref)

    st_ref[...] += upd


def _attn(nbr, q, x, cw, cb):
    grid = N // ATT_R
    row = pl.BlockSpec((ATT_R, 128), lambda i: (i, 0))
    return pl.pallas_call(
        _attn_body,
        grid=(grid,),
        in_specs=[
            pl.BlockSpec((ATT_R, K, 256), lambda i: (i, 0, 0)),
            row, row,
            pl.BlockSpec((128, 128), lambda i: (0, 0)),
            pl.BlockSpec((1, 128), lambda i: (0, 0)),
        ],
        out_specs=[row, pl.BlockSpec((8, 128), lambda i: (0, 0))],
        out_shape=[jax.ShapeDtypeStruct((N, 128), jnp.float32),
                   jax.ShapeDtypeStruct((8, 128), jnp.float32)],
    )(nbr, q, x, cw, cb.reshape(1, 128))


def _bn_core(h, x, st_ref, g_ref, b_ref):
    inv_n = jnp.float32(1.0 / N)
    mu = st_ref[0:1, :] * inv_n
    var = st_ref[1:2, :] * inv_n - mu * mu
    hn = (h - mu) * lax.rsqrt(var + 1e-5) * g_ref[...] + b_ref[...]
    return x + jnp.maximum(hn, 0.0)


def _bn_qkv_body(h_ref, x_ref, st_ref, g_ref, b_ref,
                 qw_ref, qb_ref, kw_ref, kb_ref, vw_ref, vb_ref,
                 out_ref, q_ref, kv_ref):
    out = _bn_core(h_ref[...], x_ref[...], st_ref, g_ref, b_ref)
    out_ref[...] = out
    dn = (((1,), (1,)), ((), ()))
    q = lax.dot_general(out, qw_ref[...], dn,
                        preferred_element_type=jnp.float32) + qb_ref[...]
    k = lax.dot_general(out, kw_ref[...], dn,
                        preferred_element_type=jnp.float32) + kb_ref[...]
    v = lax.dot_general(out, vw_ref[...], dn,
                        preferred_element_type=jnp.float32) + vb_ref[...]
    q_ref[...] = q
    kv_ref[...] = jnp.concatenate([k, v], axis=1)


def _bn_qkv(h, x, st, g, b, qw, qb, kw, kb, vw, vb):
    grid = N // ROW_R
    row = pl.BlockSpec((ROW_R, 128), lambda i: (i, 0))
    wsp = pl.BlockSpec((128, 128), lambda i: (0, 0))
    bsp = pl.BlockSpec((1, 128), lambda i: (0, 0))
    ssp = pl.BlockSpec((8, 128), lambda i: (0, 0))
    return pl.pallas_call(
        _bn_qkv_body,
        grid=(grid,),
        in_specs=[row, row, ssp, bsp, bsp, wsp, bsp, wsp, bsp, wsp, bsp],
        out_specs=[row, row, pl.BlockSpec((ROW_R, 256), lambda i: (i, 0))],
        out_shape=[jax.ShapeDtypeStruct((N, 128), jnp.float32),
                   jax.ShapeDtypeStruct((N, 128), jnp.float32),
                   jax.ShapeDtypeStruct((N, 256), jnp.float32)],
    )(h, x, st, g.reshape(1, 128), b.reshape(1, 128),
      qw, qb.reshape(1, 128), kw, kb.reshape(1, 128), vw, vb.reshape(1, 128))


def _bn_final_body(h_ref, x_ref, st_ref, g_ref, b_ref, out_ref):
    out_ref[...] = _bn_core(h_ref[...], x_ref[...], st_ref, g_ref, b_ref)


def _bn_final(h, x, st, g, b):
    grid = N // ROW_R
    row = pl.BlockSpec((ROW_R, 128), lambda i: (i, 0))
    bsp = pl.BlockSpec((1, 128), lambda i: (0, 0))
    ssp = pl.BlockSpec((8, 128), lambda i: (0, 0))
    return pl.pallas_call(
        _bn_final_body,
        grid=(grid,),
        in_specs=[row, row, ssp, bsp, bsp],
        out_specs=row,
        out_shape=jax.ShapeDtypeStruct((N, 128), jnp.float32),
    )(h, x, st, g.reshape(1, 128), b.reshape(1, 128))


def kernel(coords, feats,
           q_w1, q_b1, k_w1, k_b1, v_w1, v_b1, conv_w1, conv_b1, bn_g1, bn_b1,
           q_w2, q_b2, k_w2, k_b2, v_w2, v_b2, conv_w2, conv_b2, bn_g2, bn_b2):
    idx = _knn(coords)
    idx_flat = idx.reshape(N * K)

    q1, kv1 = _qkv(feats, q_w1, q_b1, k_w1, k_b1, v_w1, v_b1)
    nbr1 = _sc_gather(idx_flat, kv1).reshape(N, K, 256)
    h1, st1 = _attn(nbr1, q1, feats, conv_w1, conv_b1)
    out1, q2, kv2 = _bn_qkv(h1, feats, st1, bn_g1, bn_b1,
                            q_w2, q_b2, k_w2, k_b2, v_w2, v_b2)

    nbr2 = _sc_gather(idx_flat, kv2).reshape(N, K, 256)
    h2, st2 = _attn(nbr2, q2, out1, conv_w2, conv_b2)
    return _bn_final(h2, out1, st2, bn_g2, bn_b2)


# speedup vs baseline: 8.1745x; 1.0407x over previous
import functools

import jax
import jax.numpy as jnp
from jax import lax
from jax.experimental import pallas as pl
from jax.experimental.pallas import tpu as pltpu
from jax.experimental.pallas import tpu_sc as plsc

N = 10000
D = 128
K = 16
NPAD = 10240
KNN_R = 80
ATT_R = 400
ROW_R = 2000
NWORK = 32
GCHUNK = 200


def _knn_body(cpad_ref, ct_ref, idx_ref, sq_ref):
    i = pl.program_id(0)

    @pl.when(i == 0)
    def _():
        ct = ct_ref[...]
        sq_ref[...] = jnp.sum(ct * ct, axis=0, keepdims=True)

    crow = cpad_ref[...]
    dot = lax.dot_general(crow * jnp.float32(-2.0), ct_ref[...],
                          dimension_numbers=(((1,), (0,)), ((), ())),
                          preferred_element_type=jnp.float32)
    d2 = sq_ref[...] + dot
    inf = jnp.float32(jnp.inf)

    C = 256
    T = 5
    G = NPAD // C
    cls = lax.broadcasted_iota(jnp.int32, (KNN_R, C), 1)
    mv = [jnp.full((KNN_R, C), inf, jnp.float32) for _ in range(T)]
    mi = [jnp.zeros((KNN_R, C), jnp.int32) for _ in range(T)]
    for g in range(G):
        x = lax.slice(d2, (0, g * C), (KNN_R, (g + 1) * C))
        xi = cls + jnp.int32(g * C)
        if (g + 1) * C > N:
            x = jnp.where(xi >= N, inf, x)
        for t in range(T):
            c = x < mv[t]
            nm = jnp.where(c, x, mv[t])
            ni = jnp.where(c, xi, mi[t])
            x = jnp.where(c, mv[t], x)
            xi = jnp.where(c, mi[t], xi)
            mv[t] = nm
            mi[t] = ni
    cand = jnp.concatenate(mv, axis=1)
    cidx = jnp.concatenate(mi, axis=1)

    picks = []
    big = jnp.int32(2**30)
    for _ in range(K):
        m = jnp.min(cand, axis=1, keepdims=True)
        sel = jnp.where(cand == m, cidx, big)
        amin = jnp.min(sel, axis=1, keepdims=True)
        picks.append(amin)
        cand = jnp.where(sel == amin, inf, cand)
    idx_ref[...] = jnp.concatenate(picks, axis=1)


def _knn(coords):
    cpad = jnp.zeros((NPAD, 128), jnp.float32).at[:N, :3].set(coords)
    ct = cpad.T
    grid = N // KNN_R
    return pl.pallas_call(
        _knn_body,
        grid=(grid,),
        in_specs=[
            pl.BlockSpec((KNN_R, 128), lambda i: (i, 0)),
            pl.BlockSpec((128, NPAD), lambda i: (0, 0)),
        ],
        out_specs=pl.BlockSpec((KNN_R, K), lambda i: (i, 0)),
        out_shape=jax.ShapeDtypeStruct((N, K), jnp.int32),
        scratch_shapes=[pltpu.VMEM((1, NPAD), jnp.float32)],
    )(cpad, ct)


def _qkv_body(x_ref, qw_ref, qb_ref, kw_ref, kb_ref, vw_ref, vb_ref,
              q_ref, kv_ref):
    x = x_ref[...]
    dn = (((1,), (1,)), ((), ()))
    q = lax.dot_general(x, qw_ref[...], dn,
                        preferred_element_type=jnp.float32) + qb_ref[...]
    k = lax.dot_general(x, kw_ref[...], dn,
                        preferred_element_type=jnp.float32) + kb_ref[...]
    v = lax.dot_general(x, vw_ref[...], dn,
                        preferred_element_type=jnp.float32) + vb_ref[...]
    q_ref[...] = q
    kv_ref[...] = jnp.concatenate([k, v], axis=1)


def _qkv(x, qw, qb, kw, kb, vw, vb):
    grid = N // ROW_R
    row = pl.BlockSpec((ROW_R, 128), lambda i: (i, 0))
    wsp = pl.BlockSpec((128, 128), lambda i: (0, 0))
    bsp = pl.BlockSpec((1, 128), lambda i: (0, 0))
    return pl.pallas_call(
        _qkv_body,
        grid=(grid,),
        in_specs=[row, wsp, bsp, wsp, bsp, wsp, bsp],
        out_specs=[row, pl.BlockSpec((ROW_R, 256), lambda i: (i, 0))],
        out_shape=[jax.ShapeDtypeStruct((N, 128), jnp.float32),
                   jax.ShapeDtypeStruct((N, 256), jnp.float32)],
    )(x, qw, qb.reshape(1, 128), kw, kb.reshape(1, 128),
      vw, vb.reshape(1, 128))


def _gather_body(idx_hbm, table_hbm, out_hbm, idx_v, rows0, rows1, sem0, sem1):
    wid = lax.axis_index("s") * 2 + lax.axis_index("c")
    per_w = (N * K) // NWORK
    nchunk = per_w // GCHUNK
    base = wid * per_w
    pltpu.sync_copy(idx_hbm.at[pl.ds(base, per_w)], idx_v)

    def start(c, buf, sem):
        pltpu.async_copy(table_hbm.at[idx_v.at[pl.ds(c * GCHUNK, GCHUNK)]],
                         buf, sem)

    def fin(c, buf, sem):
        pltpu.make_async_copy(table_hbm.at[pl.ds(0, GCHUNK)], buf, sem).wait()
        pltpu.sync_copy(buf, out_hbm.at[pl.ds(base + c * GCHUNK, GCHUNK)])

    start(0, rows0, sem0)
    fin(0, rows0, sem0)
    start(1, rows0, sem0)
    start(2, rows1, sem1)

    def pair(i, carry):
        c0 = 2 * i + 1
        fin(c0, rows0, sem0)

        @pl.when(c0 + 2 < nchunk)
        def _():
            start(c0 + 2, rows0, sem0)

        fin(c0 + 1, rows1, sem1)

        @pl.when(c0 + 3 < nchunk)
        def _():
            start(c0 + 3, rows1, sem1)

        return carry

    lax.fori_loop(0, (nchunk - 1) // 2, pair, 0)


def _sc_gather(idx_flat, table):
    per_w = (N * K) // NWORK
    mesh = plsc.VectorSubcoreMesh(core_axis_name="c", subcore_axis_name="s")
    fn = pl.kernel(
        _gather_body,
        mesh=mesh,
        out_type=jax.ShapeDtypeStruct((N * K, 256), jnp.float32),
        scratch_types=[
            pltpu.VMEM((per_w,), jnp.int32),
            pltpu.VMEM((GCHUNK, 256), jnp.float32),
            pltpu.VMEM((GCHUNK, 256), jnp.float32),
            pltpu.SemaphoreType.DMA,
            pltpu.SemaphoreType.DMA,
        ],
    )
    return fn(idx_flat, table)


def _attn_body(nbr_ref, q_ref, x_ref, cw_ref, cb_ref, h_ref, st_ref):
    i = pl.program_id(0)
    nbr = nbr_ref[...]
    nk = nbr[:, :, :128]
    nv = nbr[:, :, 128:]
    q = q_ref[...]
    logits = jnp.sum(nk * q[:, None, :], axis=2)
    logits = logits * jnp.float32(1.0 / jnp.sqrt(128.0))
    m = jnp.max(logits, axis=1, keepdims=True)
    e = jnp.exp(logits - m)
    attn = e / jnp.sum(e, axis=1, keepdims=True)
    att = jnp.sum(attn[:, :, None] * nv, axis=1)
    h = lax.dot_general(x_ref[...] - att, cw_ref[...],
                        (((1,), (1,)), ((), ())),
                        preferred_element_type=jnp.float32) + cb_ref[...]
    h_ref[...] = h
    s1 = jnp.sum(h, axis=0, keepdims=True)
    s2 = jnp.sum(h * h, axis=0, keepdims=True)
    pad = jnp.zeros((6, 128), jnp.float32)
    upd = jnp.concatenate([s1, s2, pad], axis=0)

    @pl.when(i == 0)
    def _():
        st_ref[...] = jnp.zeros_like(st_ref)

    st_ref[...] += upd


def _attn(nbr, q, x, cw, cb):
    grid = N // ATT_R
    row = pl.BlockSpec((ATT_R, 128), lambda i: (i, 0))
    return pl.pallas_call(
        _attn_body,
        grid=(grid,),
        in_specs=[
            pl.BlockSpec((ATT_R, K, 256), lambda i: (i, 0, 0)),
            row, row,
            pl.BlockSpec((128, 128), lambda i: (0, 0)),
            pl.BlockSpec((1, 128), lambda i: (0, 0)),
        ],
        out_specs=[row, pl.BlockSpec((8, 128), lambda i: (0, 0))],
        out_shape=[jax.ShapeDtypeStruct((N, 128), jnp.float32),
                   jax.ShapeDtypeStruct((8, 128), jnp.float32)],
    )(nbr, q, x, cw, cb.reshape(1, 128))


def _bn_core(h, x, st_ref, g_ref, b_ref):
    inv_n = jnp.float32(1.0 / N)
    mu = st_ref[0:1, :] * inv_n
    var = st_ref[1:2, :] * inv_n - mu * mu
    hn = (h - mu) * lax.rsqrt(var + 1e-5) * g_ref[...] + b_ref[...]
    return x + jnp.maximum(hn, 0.0)


def _bn_qkv_body(h_ref, x_ref, st_ref, g_ref, b_ref,
                 qw_ref, qb_ref, kw_ref, kb_ref, vw_ref, vb_ref,
                 out_ref, q_ref, kv_ref):
    out = _bn_core(h_ref[...], x_ref[...], st_ref, g_ref, b_ref)
    out_ref[...] = out
    dn = (((1,), (1,)), ((), ()))
    q = lax.dot_general(out, qw_ref[...], dn,
                        preferred_element_type=jnp.float32) + qb_ref[...]
    k = lax.dot_general(out, kw_ref[...], dn,
                        preferred_element_type=jnp.float32) + kb_ref[...]
    v = lax.dot_general(out, vw_ref[...], dn,
                        preferred_element_type=jnp.float32) + vb_ref[...]
    q_ref[...] = q
    kv_ref[...] = jnp.concatenate([k, v], axis=1)


def _bn_qkv(h, x, st, g, b, qw, qb, kw, kb, vw, vb):
    grid = N // ROW_R
    row = pl.BlockSpec((ROW_R, 128), lambda i: (i, 0))
    wsp = pl.BlockSpec((128, 128), lambda i: (0, 0))
    bsp = pl.BlockSpec((1, 128), lambda i: (0, 0))
    ssp = pl.BlockSpec((8, 128), lambda i: (0, 0))
    return pl.pallas_call(
        _bn_qkv_body,
        grid=(grid,),
        in_specs=[row, row, ssp, bsp, bsp, wsp, bsp, wsp, bsp, wsp, bsp],
        out_specs=[row, row, pl.BlockSpec((ROW_R, 256), lambda i: (i, 0))],
        out_shape=[jax.ShapeDtypeStruct((N, 128), jnp.float32),
                   jax.ShapeDtypeStruct((N, 128), jnp.float32),
                   jax.ShapeDtypeStruct((N, 256), jnp.float32)],
    )(h, x, st, g.reshape(1, 128), b.reshape(1, 128),
      qw, qb.reshape(1, 128), kw, kb.reshape(1, 128), vw, vb.reshape(1, 128))


def _bn_final_body(h_ref, x_ref, st_ref, g_ref, b_ref, out_ref):
    out_ref[...] = _bn_core(h_ref[...], x_ref[...], st_ref, g_ref, b_ref)


def _bn_final(h, x, st, g, b):
    grid = N // ROW_R
    row = pl.BlockSpec((ROW_R, 128), lambda i: (i, 0))
    bsp = pl.BlockSpec((1, 128), lambda i: (0, 0))
    ssp = pl.BlockSpec((8, 128), lambda i: (0, 0))
    return pl.pallas_call(
        _bn_final_body,
        grid=(grid,),
        in_specs=[row, row, ssp, bsp, bsp],
        out_specs=row,
        out_shape=jax.ShapeDtypeStruct((N, 128), jnp.float32),
    )(h, x, st, g.reshape(1, 128), b.reshape(1, 128))


def kernel(coords, feats,
           q_w1, q_b1, k_w1, k_b1, v_w1, v_b1, conv_w1, conv_b1, bn_g1, bn_b1,
           q_w2, q_b2, k_w2, k_b2, v_w2, v_b2, conv_w2, conv_b2, bn_g2, bn_b2):
    idx = _knn(coords)
    idx_flat = idx.reshape(N * K)

    q1, kv1 = _qkv(feats, q_w1, q_b1, k_w1, k_b1, v_w1, v_b1)
    nbr1 = _sc_gather(idx_flat, kv1).reshape(N, K, 256)
    h1, st1 = _attn(nbr1, q1, feats, conv_w1, conv_b1)
    out1, q2, kv2 = _bn_qkv(h1, feats, st1, bn_g1, bn_b1,
                            q_w2, q_b2, k_w2, k_b2, v_w2, v_b2)

    nbr2 = _sc_gather(idx_flat, kv2).reshape(N, K, 256)
    h2, st2 = _attn(nbr2, q2, out1, conv_w2, conv_b2)
    return _bn_final(h2, out1, st2, bn_g2, bn_b2)


# speedup vs baseline: 11.0551x; 1.3524x over previous
import functools

import jax
import jax.numpy as jnp
from jax import lax
from jax.experimental import pallas as pl
from jax.experimental.pallas import tpu as pltpu
from jax.experimental.pallas import tpu_sc as plsc

N = 10000
D = 128
K = 16
NPAD = 10240
KNN_R = 200
ATT_R = 400
ROW_R = 2000
NWORK = 32
GCHUNK = 200


def _knn_body(cpad_ref, ct_ref, idx_ref, sq_ref):
    i = pl.program_id(0)

    @pl.when(i == 0)
    def _():
        ct = ct_ref[...]
        sq_ref[...] = jnp.sum(ct * ct, axis=0, keepdims=True)

    crow = cpad_ref[...]
    dot = lax.dot_general(crow * jnp.float32(-2.0), ct_ref[...],
                          dimension_numbers=(((1,), (0,)), ((), ())),
                          preferred_element_type=jnp.float32)
    d2 = sq_ref[...] + dot
    inf = jnp.float32(jnp.inf)

    C = 256
    T = 4
    G = NPAD // C
    cls = lax.broadcasted_iota(jnp.int32, (KNN_R, C), 1)
    mv = [jnp.full((KNN_R, C), inf, jnp.float32) for _ in range(T)]
    mi = [jnp.zeros((KNN_R, C), jnp.int32) for _ in range(T)]
    for g in range(G):
        x = lax.slice(d2, (0, g * C), (KNN_R, (g + 1) * C))
        xi = cls + jnp.int32(g * C)
        if (g + 1) * C > N:
            x = jnp.where(xi >= N, inf, x)
        for t in range(T):
            c = x < mv[t]
            nm = jnp.where(c, x, mv[t])
            ni = jnp.where(c, xi, mi[t])
            x = jnp.where(c, mv[t], x)
            xi = jnp.where(c, mi[t], xi)
            mv[t] = nm
            mi[t] = ni
    cand = jnp.concatenate(mv, axis=1)
    cidx = jnp.concatenate(mi, axis=1)

    picks = []
    big = jnp.int32(2**30)
    for _ in range(K):
        m = jnp.min(cand, axis=1, keepdims=True)
        sel = jnp.where(cand == m, cidx, big)
        amin = jnp.min(sel, axis=1, keepdims=True)
        picks.append(amin)
        cand = jnp.where(sel == amin, inf, cand)
    idx_ref[...] = jnp.concatenate(picks, axis=1)


def _knn(coords):
    cpad = jnp.zeros((NPAD, 128), jnp.float32).at[:N, :3].set(coords)
    ct = cpad.T
    grid = N // KNN_R
    return pl.pallas_call(
        _knn_body,
        grid=(grid,),
        in_specs=[
            pl.BlockSpec((KNN_R, 128), lambda i: (i, 0)),
            pl.BlockSpec((128, NPAD), lambda i: (0, 0)),
        ],
        out_specs=pl.BlockSpec((KNN_R, K), lambda i: (i, 0)),
        out_shape=jax.ShapeDtypeStruct((N, K), jnp.int32),
        scratch_shapes=[pltpu.VMEM((1, NPAD), jnp.float32)],
    )(cpad, ct)


def _qkv_body(x_ref, qw_ref, qb_ref, kw_ref, kb_ref, vw_ref, vb_ref,
              q_ref, kv_ref):
    x = x_ref[...]
    dn = (((1,), (1,)), ((), ()))
    q = lax.dot_general(x, qw_ref[...], dn,
                        preferred_element_type=jnp.float32) + qb_ref[...]
    k = lax.dot_general(x, kw_ref[...], dn,
                        preferred_element_type=jnp.float32) + kb_ref[...]
    v = lax.dot_general(x, vw_ref[...], dn,
                        preferred_element_type=jnp.float32) + vb_ref[...]
    q_ref[...] = q
    kv_ref[...] = jnp.concatenate([k, v], axis=1)


def _qkv(x, qw, qb, kw, kb, vw, vb):
    grid = N // ROW_R
    row = pl.BlockSpec((ROW_R, 128), lambda i: (i, 0))
    wsp = pl.BlockSpec((128, 128), lambda i: (0, 0))
    bsp = pl.BlockSpec((1, 128), lambda i: (0, 0))
    return pl.pallas_call(
        _qkv_body,
        grid=(grid,),
        in_specs=[row, wsp, bsp, wsp, bsp, wsp, bsp],
        out_specs=[row, pl.BlockSpec((ROW_R, 256), lambda i: (i, 0))],
        out_shape=[jax.ShapeDtypeStruct((N, 128), jnp.float32),
                   jax.ShapeDtypeStruct((N, 256), jnp.float32)],
    )(x, qw, qb.reshape(1, 128), kw, kb.reshape(1, 128),
      vw, vb.reshape(1, 128))


def _gather_body(idx_hbm, table_hbm, out_hbm, idx_v, rows0, rows1, sem0, sem1):
    wid = lax.axis_index("s") * 2 + lax.axis_index("c")
    per_w = (N * K) // NWORK
    nchunk = per_w // GCHUNK
    base = wid * per_w
    pltpu.sync_copy(idx_hbm.at[pl.ds(base, per_w)], idx_v)

    def start(c, buf, sem):
        pltpu.async_copy(table_hbm.at[idx_v.at[pl.ds(c * GCHUNK, GCHUNK)]],
                         buf, sem)

    def fin(c, buf, sem):
        pltpu.make_async_copy(table_hbm.at[pl.ds(0, GCHUNK)], buf, sem).wait()
        pltpu.sync_copy(buf, out_hbm.at[pl.ds(base + c * GCHUNK, GCHUNK)])

    start(0, rows0, sem0)
    fin(0, rows0, sem0)
    start(1, rows0, sem0)
    start(2, rows1, sem1)

    def pair(i, carry):
        c0 = 2 * i + 1
        fin(c0, rows0, sem0)

        @pl.when(c0 + 2 < nchunk)
        def _():
            start(c0 + 2, rows0, sem0)

        fin(c0 + 1, rows1, sem1)

        @pl.when(c0 + 3 < nchunk)
        def _():
            start(c0 + 3, rows1, sem1)

        return carry

    lax.fori_loop(0, (nchunk - 1) // 2, pair, 0)


def _sc_gather(idx_flat, table):
    per_w = (N * K) // NWORK
    mesh = plsc.VectorSubcoreMesh(core_axis_name="c", subcore_axis_name="s")
    fn = pl.kernel(
        _gather_body,
        mesh=mesh,
        out_type=jax.ShapeDtypeStruct((N * K, 256), jnp.float32),
        scratch_types=[
            pltpu.VMEM((per_w,), jnp.int32),
            pltpu.VMEM((GCHUNK, 256), jnp.float32),
            pltpu.VMEM((GCHUNK, 256), jnp.float32),
            pltpu.SemaphoreType.DMA,
            pltpu.SemaphoreType.DMA,
        ],
    )
    return fn(idx_flat, table)


def _attn_body(nbr_ref, q_ref, x_ref, cw_ref, cb_ref, h_ref, st_ref):
    i = pl.program_id(0)
    nbr = nbr_ref[...]
    nk = nbr[:, :, :128]
    nv = nbr[:, :, 128:]
    q = q_ref[...]
    logits = jnp.sum(nk * q[:, None, :], axis=2)
    logits = logits * jnp.float32(1.0 / jnp.sqrt(128.0))
    m = jnp.max(logits, axis=1, keepdims=True)
    e = jnp.exp(logits - m)
    attn = e / jnp.sum(e, axis=1, keepdims=True)
    att = jnp.sum(attn[:, :, None] * nv, axis=1)
    h = lax.dot_general(x_ref[...] - att, cw_ref[...],
                        (((1,), (1,)), ((), ())),
                        preferred_element_type=jnp.float32) + cb_ref[...]
    h_ref[...] = h
    s1 = jnp.sum(h, axis=0, keepdims=True)
    s2 = jnp.sum(h * h, axis=0, keepdims=True)
    pad = jnp.zeros((6, 128), jnp.float32)
    upd = jnp.concatenate([s1, s2, pad], axis=0)

    @pl.when(i == 0)
    def _():
        st_ref[...] = jnp.zeros_like(st_ref)

    st_ref[...] += upd


def _attn(nbr, q, x, cw, cb):
    grid = N // ATT_R
    row = pl.BlockSpec((ATT_R, 128), lambda i: (i, 0))
    return pl.pallas_call(
        _attn_body,
        grid=(grid,),
        in_specs=[
            pl.BlockSpec((ATT_R, K, 256), lambda i: (i, 0, 0)),
            row, row,
            pl.BlockSpec((128, 128), lambda i: (0, 0)),
            pl.BlockSpec((1, 128), lambda i: (0, 0)),
        ],
        out_specs=[row, pl.BlockSpec((8, 128), lambda i: (0, 0))],
        out_shape=[jax.ShapeDtypeStruct((N, 128), jnp.float32),
                   jax.ShapeDtypeStruct((8, 128), jnp.float32)],
    )(nbr, q, x, cw, cb.reshape(1, 128))


def _bn_core(h, x, st_ref, g_ref, b_ref):
    inv_n = jnp.float32(1.0 / N)
    mu = st_ref[0:1, :] * inv_n
    var = st_ref[1:2, :] * inv_n - mu * mu
    hn = (h - mu) * lax.rsqrt(var + 1e-5) * g_ref[...] + b_ref[...]
    return x + jnp.maximum(hn, 0.0)


def _bn_qkv_body(h_ref, x_ref, st_ref, g_ref, b_ref,
                 qw_ref, qb_ref, kw_ref, kb_ref, vw_ref, vb_ref,
                 out_ref, q_ref, kv_ref):
    out = _bn_core(h_ref[...], x_ref[...], st_ref, g_ref, b_ref)
    out_ref[...] = out
    dn = (((1,), (1,)), ((), ()))
    q = lax.dot_general(out, qw_ref[...], dn,
                        preferred_element_type=jnp.float32) + qb_ref[...]
    k = lax.dot_general(out, kw_ref[...], dn,
                        preferred_element_type=jnp.float32) + kb_ref[...]
    v = lax.dot_general(out, vw_ref[...], dn,
                        preferred_element_type=jnp.float32) + vb_ref[...]
    q_ref[...] = q
    kv_ref[...] = jnp.concatenate([k, v], axis=1)


def _bn_qkv(h, x, st, g, b, qw, qb, kw, kb, vw, vb):
    grid = N // ROW_R
    row = pl.BlockSpec((ROW_R, 128), lambda i: (i, 0))
    wsp = pl.BlockSpec((128, 128), lambda i: (0, 0))
    bsp = pl.BlockSpec((1, 128), lambda i: (0, 0))
    ssp = pl.BlockSpec((8, 128), lambda i: (0, 0))
    return pl.pallas_call(
        _bn_qkv_body,
        grid=(grid,),
        in_specs=[row, row, ssp, bsp, bsp, wsp, bsp, wsp, bsp, wsp, bsp],
        out_specs=[row, row, pl.BlockSpec((ROW_R, 256), lambda i: (i, 0))],
        out_shape=[jax.ShapeDtypeStruct((N, 128), jnp.float32),
                   jax.ShapeDtypeStruct((N, 128), jnp.float32),
                   jax.ShapeDtypeStruct((N, 256), jnp.float32)],
    )(h, x, st, g.reshape(1, 128), b.reshape(1, 128),
      qw, qb.reshape(1, 128), kw, kb.reshape(1, 128), vw, vb.reshape(1, 128))


def _bn_final_body(h_ref, x_ref, st_ref, g_ref, b_ref, out_ref):
    out_ref[...] = _bn_core(h_ref[...], x_ref[...], st_ref, g_ref, b_ref)


def _bn_final(h, x, st, g, b):
    grid = N // ROW_R
    row = pl.BlockSpec((ROW_R, 128), lambda i: (i, 0))
    bsp = pl.BlockSpec((1, 128), lambda i: (0, 0))
    ssp = pl.BlockSpec((8, 128), lambda i: (0, 0))
    return pl.pallas_call(
        _bn_final_body,
        grid=(grid,),
        in_specs=[row, row, ssp, bsp, bsp],
        out_specs=row,
        out_shape=jax.ShapeDtypeStruct((N, 128), jnp.float32),
    )(h, x, st, g.reshape(1, 128), b.reshape(1, 128))


def kernel(coords, feats,
           q_w1, q_b1, k_w1, k_b1, v_w1, v_b1, conv_w1, conv_b1, bn_g1, bn_b1,
           q_w2, q_b2, k_w2, k_b2, v_w2, v_b2, conv_w2, conv_b2, bn_g2, bn_b2):
    idx = _knn(coords)
    idx_flat = idx.reshape(N * K)

    q1, kv1 = _qkv(feats, q_w1, q_b1, k_w1, k_b1, v_w1, v_b1)
    nbr1 = _sc_gather(idx_flat, kv1).reshape(N, K, 256)
    h1, st1 = _attn(nbr1, q1, feats, conv_w1, conv_b1)
    out1, q2, kv2 = _bn_qkv(h1, feats, st1, bn_g1, bn_b1,
                            q_w2, q_b2, k_w2, k_b2, v_w2, v_b2)

    nbr2 = _sc_gather(idx_flat, kv2).reshape(N, K, 256)
    h2, st2 = _attn(nbr2, q2, out1, conv_w2, conv_b2)
    return _bn_final(h2, out1, st2, bn_g2, bn_b2)
